# baseline (speedup 1.0000x reference)
import jax
import jax.numpy as jnp
from jax import lax
from jax.experimental import pallas as pl
from jax.experimental.pallas import tpu as pltpu
from jax.experimental.pallas import tpu_sc as plsc

BATCH = 16384
N_FIELDS = 26
INPUT_DIM = 1000000
NC = 2
NS = 16
NW = NC * NS
ROWS_PER_W = BATCH // NW
VALS_PER_W = ROWS_PER_W * N_FIELDS
N_GROUPS = ROWS_PER_W // 16


def _sc_body(idx_hbm, w_hbm, b_hbm, out_hbm, idx_v, vals_v, acc_v, out_v, b_s,
             isem, hsem, gsem, bsem):
    wid = lax.axis_index("s") * NC + lax.axis_index("c")
    base = wid * ROWS_PER_W

    HEAD = 4

    def stage(f, sem):
        return pltpu.make_async_copy(
            idx_hbm.at[f, pl.ds(base, ROWS_PER_W)],
            idx_v.at[pl.ds(f * ROWS_PER_W, ROWS_PER_W)],
            sem,
        )

    for f in range(HEAD):
        stage(f, hsem).start()
    for f in range(HEAD, N_FIELDS):
        stage(f, isem).start()
    bcopy = pltpu.make_async_copy(b_hbm, b_s.at[pl.ds(0, 1)], bsem)
    bcopy.start()

    wrow = w_hbm.at[0]

    def gather(f):
        return pltpu.make_async_copy(
            wrow.at[idx_v.at[pl.ds(f * ROWS_PER_W, ROWS_PER_W)]],
            vals_v.at[pl.ds(f * ROWS_PER_W, ROWS_PER_W)],
            gsem.at[f],
        )

    pltpu.make_async_copy(
        idx_hbm.at[0, pl.ds(0, HEAD * ROWS_PER_W)],
        idx_v.at[pl.ds(0, HEAD * ROWS_PER_W)],
        hsem,
    ).wait()
    for f in range(HEAD):
        gather(f).start()
    pltpu.make_async_copy(
        idx_hbm.at[0, pl.ds(0, (N_FIELDS - HEAD) * ROWS_PER_W)],
        idx_v.at[pl.ds(0, (N_FIELDS - HEAD) * ROWS_PER_W)],
        isem,
    ).wait()
    for f in range(HEAD, N_FIELDS):
        gather(f).start()

    bcopy.wait()
    bias = plsc.load_gather(b_s, [lax.iota(jnp.int32, 16) * 0])

    gather(0).wait()

    def init_group(g, carry):
        off = pl.multiple_of(g * 16, 16)
        acc_v[pl.ds(off, 16)] = bias + vals_v[pl.ds(off, 16)]
        return carry

    lax.fori_loop(0, N_GROUPS, init_group, 0)

    for f in range(1, N_FIELDS - 1):
        gather(f).wait()

        def add_group(g, carry, f=f):
            off = pl.multiple_of(g * 16, 16)
            plsc.addupdate(
                acc_v.at[pl.ds(off, 16)],
                vals_v[pl.ds(f * ROWS_PER_W + off, 16)],
            )
            return carry

        lax.fori_loop(0, N_GROUPS, add_group, 0)

    gather(N_FIELDS - 1).wait()

    def sig_group(g, carry):
        off = pl.multiple_of(g * 16, 16)
        acc = acc_v[pl.ds(off, 16)] + vals_v[
            pl.ds((N_FIELDS - 1) * ROWS_PER_W + off, 16)
        ]
        out_v[pl.ds(off, 16)] = 1.0 / (1.0 + jnp.exp(-acc))
        return carry

    lax.fori_loop(0, N_GROUPS, sig_group, 0)
    pltpu.sync_copy(out_v, out_hbm.at[pl.ds(base, ROWS_PER_W)])


@jax.jit
def kernel(indices, w, b):
    idx_t = indices.astype(jnp.int32).T
    wf = w.T

    run = pl.kernel(
        _sc_body,
        out_type=jax.ShapeDtypeStruct((BATCH,), jnp.float32),
        mesh=plsc.VectorSubcoreMesh(core_axis_name="c", subcore_axis_name="s"),
        compiler_params=pltpu.CompilerParams(needs_layout_passes=False),
        scratch_types=[
            pltpu.VMEM((VALS_PER_W,), jnp.int32),
            pltpu.VMEM((VALS_PER_W,), jnp.float32),
            pltpu.VMEM((ROWS_PER_W,), jnp.float32),
            pltpu.VMEM((ROWS_PER_W,), jnp.float32),
            pltpu.VMEM((16,), jnp.float32),
            pltpu.SemaphoreType.DMA,
            pltpu.SemaphoreType.DMA,
            pltpu.SemaphoreType.DMA((N_FIELDS,)),
            pltpu.SemaphoreType.DMA,
        ],
    )
    return run(idx_t, wf, b)
